# Initial kernel scaffold; baseline (speedup 1.0000x reference)
import functools

import jax
import jax.numpy as jnp
from jax import lax
from jax.experimental import pallas as pl
from jax.experimental.pallas import tpu as pltpu
from jax.experimental.pallas import tpu_sc as plsc

N = 10000
E = 320000
D = 128
NC = 2
NS = 16
NW = NC * NS
EPW = E // NW
BLK = 80
NBLK = EPW // BLK
RPT = N // NS
ZR = 125
DEGW = 16


def _sc_body(with_deg, *refs):
    if with_deg:
        (h, src, dst, zrows, zdeg, ones16,
         aggp, degp, agg_sh, deg_sh, zbuf, dzbuf, ones, rows, sidx, didx, sem) = refs
    else:
        (h, src, dst, zrows,
         aggp, agg_sh, zbuf, rows, sidx, didx, sem) = refs

    c = lax.axis_index("c")
    s = lax.axis_index("s")
    w = s * NC + c

    pltpu.sync_copy(zrows, zbuf)
    for k in range(RPT // ZR):
        pltpu.sync_copy(zbuf, agg_sh.at[pl.ds(s * RPT + k * ZR, ZR)])
    if with_deg:
        pltpu.sync_copy(zdeg, dzbuf)
        pltpu.sync_copy(ones16, ones)
        pltpu.sync_copy(dzbuf, deg_sh.at[pl.ds(s * RPT, RPT)])
    plsc.subcore_barrier()

    def blk(i, carry):
        base = w * EPW + i * BLK
        pltpu.sync_copy(src.at[pl.ds(base, BLK)], sidx)
        pltpu.sync_copy(dst.at[pl.ds(base, BLK)], didx)
        pltpu.async_copy(h.at[sidx], rows, sem).wait()
        pltpu.sync_copy(rows, agg_sh.at[didx], add=True)
        if with_deg:
            pltpu.sync_copy(ones, deg_sh.at[didx], add=True)
        return carry

    lax.fori_loop(0, NBLK, blk, 0)
    plsc.subcore_barrier()

    for k in range(RPT // ZR):
        row = s * RPT + k * ZR
        pltpu.sync_copy(agg_sh.at[pl.ds(row, ZR)], zbuf)
        pltpu.sync_copy(zbuf, aggp.at[c, pl.ds(row, ZR)])
    if with_deg:
        pltpu.sync_copy(deg_sh.at[pl.ds(s * RPT, RPT)], dzbuf)
        pltpu.sync_copy(dzbuf, degp.at[c, pl.ds(s * RPT, RPT)])


@functools.cache
def _make_sc_agg(with_deg):
    mesh = plsc.VectorSubcoreMesh(core_axis_name="c", subcore_axis_name="s")
    out_type = [jax.ShapeDtypeStruct((NC, N, D), jnp.float32)]
    scratch = [
        pltpu.VMEM_SHARED((N, D), jnp.float32),
    ]
    if with_deg:
        out_type.append(jax.ShapeDtypeStruct((NC, N, DEGW), jnp.float32))
        scratch.append(pltpu.VMEM_SHARED((N, DEGW), jnp.float32))
    scratch += [pltpu.VMEM((ZR, D), jnp.float32)]
    if with_deg:
        scratch += [
            pltpu.VMEM((RPT, DEGW), jnp.float32),
            pltpu.VMEM((BLK, DEGW), jnp.float32),
        ]
    scratch += [
        pltpu.VMEM((BLK, D), jnp.float32),
        pltpu.VMEM((BLK,), jnp.int32),
        pltpu.VMEM((BLK,), jnp.int32),
        pltpu.SemaphoreType.DMA,
    ]
    return pl.kernel(
        functools.partial(_sc_body, with_deg),
        out_type=out_type,
        mesh=mesh,
        scratch_types=scratch,
    )


def _tc_body(relu, aggp_ref, degp_ref, h_ref, wl_ref, wr_ref, b_ref, out_ref):
    agg = aggp_ref[0] + aggp_ref[1]
    deg = degp_ref[0, :, 0:1] + degp_ref[1, :, 0:1]
    inv = 1.0 / jnp.maximum(deg, 1.0)
    dn = (((1,), (1,)), ((), ()))
    acc = lax.dot_general(agg * inv, wl_ref[...], dn,
                          preferred_element_type=jnp.float32,
                          precision=lax.Precision.HIGHEST)
    acc = acc + lax.dot_general(h_ref[...], wr_ref[...], dn,
                                preferred_element_type=jnp.float32,
                                precision=lax.Precision.HIGHEST)
    acc = acc + b_ref[...]
    if relu:
        acc = jnp.maximum(acc, 0.0)
    out_ref[...] = acc


@functools.cache
def _make_tc_layer(relu):
    BM = 1000
    return pl.pallas_call(
        functools.partial(_tc_body, relu),
        grid=(N // BM,),
        in_specs=[
            pl.BlockSpec((NC, BM, D), lambda i: (0, i, 0)),
            pl.BlockSpec((NC, BM, DEGW), lambda i: (0, i, 0)),
            pl.BlockSpec((BM, D), lambda i: (i, 0)),
            pl.BlockSpec((D, D), lambda i: (0, 0)),
            pl.BlockSpec((D, D), lambda i: (0, 0)),
            pl.BlockSpec((1, D), lambda i: (0, 0)),
        ],
        out_specs=pl.BlockSpec((BM, D), lambda i: (i, 0)),
        out_shape=jax.ShapeDtypeStruct((N, D), jnp.float32),
    )


def kernel(x, edge_index, W1l, W1r, W2l, W2r, W3l, W3r, b1, b2, b3):
    src = edge_index[0]
    dst = edge_index[1]
    zrows = jnp.zeros((ZR, D), jnp.float32)
    zdeg = jnp.zeros((RPT, DEGW), jnp.float32)
    ones16 = jnp.ones((BLK, DEGW), jnp.float32)

    aggp1, degp = _make_sc_agg(True)(x, src, dst, zrows, zdeg, ones16)
    h1 = _make_tc_layer(True)(aggp1, degp, x, W1l, W1r, b1.reshape(1, D))
    aggp2 = _make_sc_agg(False)(h1, src, dst, zrows)
    h2 = _make_tc_layer(True)(aggp2, degp, h1, W2l, W2r, b2.reshape(1, D))
    aggp3 = _make_sc_agg(False)(h2, src, dst, zrows)
    h3 = _make_tc_layer(False)(aggp3, degp, h2, W3l, W3r, b3.reshape(1, D))
    return h3


# speedup vs baseline: 4.6871x; 4.6871x over previous
import functools

import jax
import jax.numpy as jnp
from jax import lax
from jax.experimental import pallas as pl
from jax.experimental.pallas import tpu as pltpu
from jax.experimental.pallas import tpu_sc as plsc

N = 10000
NP = 10240
E = 320000
D = 128
NC = 2
NS = 16
NW = NC * NS
EPW = E // NW
BLK = 80
NBLK = EPW // BLK
RPT = NP // NS
ZR = 128
DEGW = 16


def _sc_body(with_deg, *refs):
    if with_deg:
        (h, src, dst, zrows, zdeg, ones16,
         aggp, degp, agg_sh, deg_sh, zbuf, dzbuf, ones, rows, sidx, didx, sem) = refs
    else:
        (h, src, dst, zrows,
         aggp, agg_sh, zbuf, rows, sidx, didx, sem) = refs

    c = lax.axis_index("c")
    s = lax.axis_index("s")
    w = s * NC + c

    pltpu.sync_copy(zrows, zbuf)
    for k in range(RPT // ZR):
        pltpu.sync_copy(zbuf, agg_sh.at[pl.ds(s * RPT + k * ZR, ZR)])
    if with_deg:
        pltpu.sync_copy(zdeg, dzbuf)
        pltpu.sync_copy(ones16, ones)
        pltpu.sync_copy(dzbuf, deg_sh.at[pl.ds(s * RPT, RPT)])
    plsc.subcore_barrier()

    def blk(i, carry):
        base = w * EPW + i * BLK
        pltpu.sync_copy(src.at[pl.ds(base, BLK)], sidx)
        pltpu.sync_copy(dst.at[pl.ds(base, BLK)], didx)
        pltpu.async_copy(h.at[sidx], rows, sem).wait()
        pltpu.sync_copy(rows, agg_sh.at[didx], add=True)
        if with_deg:
            pltpu.sync_copy(ones, deg_sh.at[didx], add=True)
        return carry

    lax.fori_loop(0, NBLK, blk, 0)
    plsc.subcore_barrier()

    for k in range(RPT // ZR):
        row = s * RPT + k * ZR
        pltpu.sync_copy(agg_sh.at[pl.ds(row, ZR)], zbuf)
        pltpu.sync_copy(zbuf, aggp.at[c, pl.ds(row, ZR)])
    if with_deg:
        pltpu.sync_copy(deg_sh.at[pl.ds(s * RPT, RPT)], dzbuf)
        pltpu.sync_copy(dzbuf, degp.at[c, pl.ds(s * RPT, RPT)])


@functools.cache
def _make_sc_agg(with_deg):
    mesh = plsc.VectorSubcoreMesh(core_axis_name="c", subcore_axis_name="s",
                                  num_cores=NC, num_subcores=NS)
    out_type = [jax.ShapeDtypeStruct((NC, NP, D), jnp.float32)]
    scratch = [
        pltpu.VMEM_SHARED((NP, D), jnp.float32),
    ]
    if with_deg:
        out_type.append(jax.ShapeDtypeStruct((NC, NP, DEGW), jnp.float32))
        scratch.append(pltpu.VMEM_SHARED((NP, DEGW), jnp.float32))
    scratch += [pltpu.VMEM((ZR, D), jnp.float32)]
    if with_deg:
        scratch += [
            pltpu.VMEM((RPT, DEGW), jnp.float32),
            pltpu.VMEM((BLK, DEGW), jnp.float32),
        ]
    scratch += [
        pltpu.VMEM((BLK, D), jnp.float32),
        pltpu.VMEM((BLK,), jnp.int32),
        pltpu.VMEM((BLK,), jnp.int32),
        pltpu.SemaphoreType.DMA,
    ]
    return pl.kernel(
        functools.partial(_sc_body, with_deg),
        out_type=out_type,
        mesh=mesh,
        scratch_types=scratch,
        compiler_params=pltpu.CompilerParams(use_tc_tiling_on_sc=False),
    )


def _tc_body(relu, aggp_ref, degp_ref, h_ref, wl_ref, wr_ref, b_ref, out_ref):
    agg = aggp_ref[0] + aggp_ref[1]
    deg = degp_ref[0, :, 0:1] + degp_ref[1, :, 0:1]
    inv = 1.0 / jnp.maximum(deg, 1.0)
    dn = (((1,), (1,)), ((), ()))
    acc = lax.dot_general(agg * inv, wl_ref[...], dn,
                          preferred_element_type=jnp.float32,
                          precision=lax.Precision.HIGHEST)
    acc = acc + lax.dot_general(h_ref[...], wr_ref[...], dn,
                                preferred_element_type=jnp.float32,
                                precision=lax.Precision.HIGHEST)
    acc = acc + b_ref[...]
    if relu:
        acc = jnp.maximum(acc, 0.0)
    out_ref[...] = acc


@functools.cache
def _make_tc_layer(relu):
    BM = 1024
    return pl.pallas_call(
        functools.partial(_tc_body, relu),
        grid=(NP // BM,),
        in_specs=[
            pl.BlockSpec((NC, BM, D), lambda i: (0, i, 0)),
            pl.BlockSpec((NC, BM, DEGW), lambda i: (0, i, 0)),
            pl.BlockSpec((BM, D), lambda i: (i, 0)),
            pl.BlockSpec((D, D), lambda i: (0, 0)),
            pl.BlockSpec((D, D), lambda i: (0, 0)),
            pl.BlockSpec((1, D), lambda i: (0, 0)),
        ],
        out_specs=pl.BlockSpec((BM, D), lambda i: (i, 0)),
        out_shape=jax.ShapeDtypeStruct((NP, D), jnp.float32),
    )


def kernel(x, edge_index, W1l, W1r, W2l, W2r, W3l, W3r, b1, b2, b3):
    src = edge_index[0]
    dst = edge_index[1]
    x = jnp.concatenate([x, jnp.zeros((NP - N, D), jnp.float32)], axis=0)
    zrows = jnp.zeros((ZR, D), jnp.float32)
    zdeg = jnp.zeros((RPT, DEGW), jnp.float32)
    ones16 = jnp.ones((BLK, DEGW), jnp.float32)

    aggp1, degp = _make_sc_agg(True)(x, src, dst, zrows, zdeg, ones16)
    h1 = _make_tc_layer(True)(aggp1, degp, x, W1l, W1r, b1.reshape(1, D))
    aggp2, = _make_sc_agg(False)(h1, src, dst, zrows)
    h2 = _make_tc_layer(True)(aggp2, degp, h1, W2l, W2r, b2.reshape(1, D))
    aggp3, = _make_sc_agg(False)(h2, src, dst, zrows)
    h3 = _make_tc_layer(False)(aggp3, degp, h2, W3l, W3r, b3.reshape(1, D))
    return h3[:N]


# speedup vs baseline: 7.4158x; 1.5822x over previous
import functools

import jax
import jax.numpy as jnp
from jax import lax
from jax.experimental import pallas as pl
from jax.experimental.pallas import tpu as pltpu
from jax.experimental.pallas import tpu_sc as plsc

N = 10000
NP = 10240
E = 320000
D = 128
NC = 2
NS = 16
NW = NC * NS
EPW = E // NW
BLK = 80
NBLK = EPW // BLK
RPT = NP // NS
ZR = 128
DEGW = 8


def _sc_body(with_deg, *refs):
    if with_deg:
        (h, src, dst, zrows, zdeg, ones16,
         aggp, degp, agg_sh, deg_sh, zbuf, dzbuf, ones,
         rows0, rows1, sidx0, sidx1, didx0, didx1, sem0, sem1) = refs
    else:
        (h, src, dst, zrows,
         aggp, agg_sh, zbuf,
         rows0, rows1, sidx0, sidx1, didx0, didx1, sem0, sem1) = refs
    rows_b = (rows0, rows1)
    sidx_b = (sidx0, sidx1)
    didx_b = (didx0, didx1)
    sem_b = (sem0, sem1)

    c = lax.axis_index("c")
    s = lax.axis_index("s")
    w = s * NC + c
    ebase = w * EPW

    pltpu.sync_copy(zrows, zbuf)
    for k in range(RPT // ZR):
        pltpu.sync_copy(zbuf, agg_sh.at[pl.ds(s * RPT + k * ZR, ZR)])
    if with_deg:
        pltpu.sync_copy(zdeg, dzbuf)
        pltpu.sync_copy(ones16, ones)
        pltpu.sync_copy(dzbuf, deg_sh.at[pl.ds(s * RPT, RPT)])
    plsc.subcore_barrier()

    def start_gather(i, b):
        base = ebase + i * BLK
        pltpu.sync_copy(src.at[pl.ds(base, BLK)], sidx_b[b])
        pltpu.sync_copy(dst.at[pl.ds(base, BLK)], didx_b[b])
        pltpu.async_copy(h.at[sidx_b[b]], rows_b[b], sem_b[b])

    def finish_block(b):
        pltpu.make_async_copy(h.at[sidx_b[b]], rows_b[b], sem_b[b]).wait()
        pltpu.sync_copy(rows_b[b], agg_sh.at[didx_b[b]], add=True)
        if with_deg:
            pltpu.sync_copy(ones, deg_sh.at[didx_b[b]], add=True)

    start_gather(0, 0)

    def pair(j, carry):
        for b in range(2):
            i = 2 * j + b

            @pl.when(i < NBLK)
            def _():
                @pl.when(i + 1 < NBLK)
                def _():
                    start_gather(i + 1, 1 - b)

                finish_block(b)
        return carry

    lax.fori_loop(0, (NBLK + 1) // 2, pair, 0)
    plsc.subcore_barrier()

    for k in range(RPT // ZR):
        row = s * RPT + k * ZR
        pltpu.sync_copy(agg_sh.at[pl.ds(row, ZR)], zbuf)
        pltpu.sync_copy(zbuf, aggp.at[c, pl.ds(row, ZR)])
    if with_deg:
        pltpu.sync_copy(deg_sh.at[pl.ds(s * RPT, RPT)], dzbuf)
        pltpu.sync_copy(dzbuf, degp.at[c, pl.ds(s * RPT, RPT)])


@functools.cache
def _make_sc_agg(with_deg):
    mesh = plsc.VectorSubcoreMesh(core_axis_name="c", subcore_axis_name="s",
                                  num_cores=NC, num_subcores=NS)
    out_type = [jax.ShapeDtypeStruct((NC, NP, D), jnp.float32)]
    scratch = [
        pltpu.VMEM_SHARED((NP, D), jnp.float32),
    ]
    if with_deg:
        out_type.append(jax.ShapeDtypeStruct((NC, NP, DEGW), jnp.float32))
        scratch.append(pltpu.VMEM_SHARED((NP, DEGW), jnp.float32))
    scratch += [pltpu.VMEM((ZR, D), jnp.float32)]
    if with_deg:
        scratch += [
            pltpu.VMEM((RPT, DEGW), jnp.float32),
            pltpu.VMEM((BLK, DEGW), jnp.float32),
        ]
    scratch += [
        pltpu.VMEM((BLK, D), jnp.float32),
        pltpu.VMEM((BLK, D), jnp.float32),
        pltpu.VMEM((BLK,), jnp.int32),
        pltpu.VMEM((BLK,), jnp.int32),
        pltpu.VMEM((BLK,), jnp.int32),
        pltpu.VMEM((BLK,), jnp.int32),
        pltpu.SemaphoreType.DMA,
        pltpu.SemaphoreType.DMA,
    ]
    return pl.kernel(
        functools.partial(_sc_body, with_deg),
        out_type=out_type,
        mesh=mesh,
        scratch_types=scratch,
        compiler_params=pltpu.CompilerParams(use_tc_tiling_on_sc=False),
    )


def _tc_body(relu, aggp_ref, degp_ref, h_ref, wl_ref, wr_ref, b_ref, out_ref):
    agg = aggp_ref[0] + aggp_ref[1]
    deg = degp_ref[0, :, 0:1] + degp_ref[1, :, 0:1]
    dn = (((1,), (1,)), ((), ()))
    acc = lax.dot_general(agg / jnp.maximum(deg, 1.0), wl_ref[...], dn,
                          preferred_element_type=jnp.float32,
                          precision=lax.Precision.HIGHEST)
    acc = acc + lax.dot_general(h_ref[...], wr_ref[...], dn,
                                preferred_element_type=jnp.float32,
                                precision=lax.Precision.HIGHEST)
    acc = acc + b_ref[...]
    if relu:
        acc = jnp.maximum(acc, 0.0)
    out_ref[...] = acc


@functools.cache
def _make_tc_layer(relu):
    BM = 1024
    return pl.pallas_call(
        functools.partial(_tc_body, relu),
        grid=(NP // BM,),
        in_specs=[
            pl.BlockSpec((NC, BM, D), lambda i: (0, i, 0)),
            pl.BlockSpec((NC, BM, DEGW), lambda i: (0, i, 0)),
            pl.BlockSpec((BM, D), lambda i: (i, 0)),
            pl.BlockSpec((D, D), lambda i: (0, 0)),
            pl.BlockSpec((D, D), lambda i: (0, 0)),
            pl.BlockSpec((1, D), lambda i: (0, 0)),
        ],
        out_specs=pl.BlockSpec((BM, D), lambda i: (i, 0)),
        out_shape=jax.ShapeDtypeStruct((NP, D), jnp.float32),
    )


def kernel(x, edge_index, W1l, W1r, W2l, W2r, W3l, W3r, b1, b2, b3):
    src = edge_index[0]
    dst = edge_index[1]
    x = jnp.concatenate([x, jnp.zeros((NP - N, D), jnp.float32)], axis=0)
    zrows = jnp.zeros((ZR, D), jnp.float32)
    zdeg = jnp.zeros((RPT, DEGW), jnp.float32)
    ones16 = jnp.ones((BLK, DEGW), jnp.float32)

    aggp1, degp = _make_sc_agg(True)(x, src, dst, zrows, zdeg, ones16)
    h1 = _make_tc_layer(True)(aggp1, degp, x, W1l, W1r, b1.reshape(1, D))
    aggp2, = _make_sc_agg(False)(h1, src, dst, zrows)
    h2 = _make_tc_layer(True)(aggp2, degp, h1, W2l, W2r, b2.reshape(1, D))
    aggp3, = _make_sc_agg(False)(h2, src, dst, zrows)
    h3 = _make_tc_layer(False)(aggp3, degp, h2, W3l, W3r, b3.reshape(1, D))
    return h3[:N]
